# baseline (speedup 1.0000x reference)
import functools

import jax
import jax.numpy as jnp
from jax import lax
from jax.experimental import pallas as pl
from jax.experimental.pallas import tpu as pltpu
from jax.experimental.pallas import tpu_sc as plsc

N = 10000
NPAD = 10240
E = 160000
C = 256
H = 128
TILES = 16
EPT = E // TILES
CHUNK = 80
SUPER = 2000
NSUPER = EPT // SUPER
NCHUNK = SUPER // CHUNK
RPT = NPAD // TILES
ZROWS = 40


def _ew_kernel(eat_ref, w1_ref, b1_ref, w2_ref, b2_ref, o_ref):
    t = jnp.dot(w1_ref[...], eat_ref[...], preferred_element_type=jnp.float32)
    t = jnp.maximum(t + b1_ref[...], 0.0)
    v = jnp.sum(t * w2_ref[...], axis=0, keepdims=True)
    o_ref[...] = jnp.maximum(v + b2_ref[...], 0.0)


def _edge_weights(edge_attr, W1, b1, W2, b2):
    be = 6400
    return pl.pallas_call(
        _ew_kernel, grid=(E // be,),
        in_specs=[pl.BlockSpec((16, be), lambda i: (0, i)),
                  pl.BlockSpec((C, 16), lambda i: (0, 0)),
                  pl.BlockSpec((C, 1), lambda i: (0, 0)),
                  pl.BlockSpec((C, 1), lambda i: (0, 0)),
                  pl.BlockSpec((1, 1), lambda i: (0, 0))],
        out_specs=pl.BlockSpec((1, be), lambda i: (0, i)),
        out_shape=jax.ShapeDtypeStruct((1, E), jnp.float32),
    )(edge_attr.T, W1, b1[:, None], W2.T, b2[None, :])


def _pre1_kernel(x_ref, w_ref, dinv_ref, h_ref, hs_ref):
    h = jnp.dot(x_ref[...], w_ref[...], preferred_element_type=jnp.float32)
    h_ref[...] = h
    hs = h * dinv_ref[...]
    hs_ref[0] = hs[:, :H]
    hs_ref[1] = hs[:, H:]


def _pre1(x, Wt, dinv):
    bm = 2000
    return pl.pallas_call(
        _pre1_kernel, grid=(N // bm,),
        in_specs=[pl.BlockSpec((bm, C), lambda i: (i, 0)),
                  pl.BlockSpec((C, C), lambda i: (0, 0)),
                  pl.BlockSpec((bm, 1), lambda i: (i, 0))],
        out_specs=[pl.BlockSpec((bm, C), lambda i: (i, 0)),
                   pl.BlockSpec((2, bm, H), lambda i: (0, i, 0))],
        out_shape=[jax.ShapeDtypeStruct((N, C), jnp.float32),
                   jax.ShapeDtypeStruct((2, N, H), jnp.float32)],
    )(x, Wt, dinv)


def _pre2_kernel(a_ref, hp_ref, dinv_ref, b_ref, w_ref, h_ref, hs_ref):
    dinv = dinv_ref[...]
    a = jnp.concatenate([a_ref[0], a_ref[1]], axis=1)
    x2 = jnp.maximum(a * dinv + hp_ref[...] * (dinv * dinv) + b_ref[...], 0.0)
    h = jnp.dot(x2, w_ref[...], preferred_element_type=jnp.float32)
    h_ref[...] = h
    hs = h * dinv
    hs_ref[0] = hs[:, :H]
    hs_ref[1] = hs[:, H:]


def _pre2(acc, hprev, dinv, b, Wt):
    bm = 2000
    return pl.pallas_call(
        _pre2_kernel, grid=(N // bm,),
        in_specs=[pl.BlockSpec((2, bm, H), lambda i: (0, i, 0)),
                  pl.BlockSpec((bm, C), lambda i: (i, 0)),
                  pl.BlockSpec((bm, 1), lambda i: (i, 0)),
                  pl.BlockSpec((1, C), lambda i: (0, 0)),
                  pl.BlockSpec((C, C), lambda i: (0, 0))],
        out_specs=[pl.BlockSpec((bm, C), lambda i: (i, 0)),
                   pl.BlockSpec((2, bm, H), lambda i: (0, i, 0))],
        out_shape=[jax.ShapeDtypeStruct((N, C), jnp.float32),
                   jax.ShapeDtypeStruct((2, N, H), jnp.float32)],
    )(acc, hprev, dinv, b[None, :], Wt)


def _final_kernel(a_ref, hp_ref, dinv_ref, bc_ref, w_ref, bl_ref, o_ref):
    dinv = dinv_ref[...]
    a = jnp.concatenate([a_ref[0], a_ref[1]], axis=1)
    x3 = jnp.maximum(a * dinv + hp_ref[...] * (dinv * dinv) + bc_ref[...], 0.0)
    o_ref[...] = jnp.dot(x3, w_ref[...], preferred_element_type=jnp.float32) + bl_ref[...]


def _final(acc, hprev, dinv, bconv, Wt, bl):
    bm = 2000
    return pl.pallas_call(
        _final_kernel, grid=(N // bm,),
        in_specs=[pl.BlockSpec((2, bm, H), lambda i: (0, i, 0)),
                  pl.BlockSpec((bm, C), lambda i: (i, 0)),
                  pl.BlockSpec((bm, 1), lambda i: (i, 0)),
                  pl.BlockSpec((1, C), lambda i: (0, 0)),
                  pl.BlockSpec((C, C), lambda i: (0, 0)),
                  pl.BlockSpec((1, C), lambda i: (0, 0))],
        out_specs=pl.BlockSpec((bm, C), lambda i: (i, 0)),
        out_shape=jax.ShapeDtypeStruct((N, C), jnp.float32),
    )(acc, hprev, dinv, bconv[None, :], Wt, bl[None, :])


def _scale_chunk(rows, ew_v, i):
    @pl.loop(0, CHUNK // 16, unroll=5)
    def _scale(g):
        wvec = ew_v[pl.ds(i * CHUNK + g * 16, 16)]
        for k in range(16):
            w = lax.gather(
                wvec, jnp.full((16, 1), k, jnp.int32),
                lax.GatherDimensionNumbers(offset_dims=(),
                                           collapsed_slice_dims=(0,),
                                           start_index_map=(0,)),
                (1,), mode=lax.GatherScatterMode.PROMISE_IN_BOUNDS)
            r = g * 16 + k
            for j in range(H // 16):
                rows[r, pl.ds(j * 16, 16)] = rows[r, pl.ds(j * 16, 16)] * w


def _sc_body(hs_hbm, src_hbm, dst_hbm, ew_hbm, zz_hbm, acc_hbm,
             acc_sh, src_v, dst_v, ew_v, bufA, bufB, bufC,
             gA, gB, gC, sA, sB, sC):
    c = lax.axis_index("c")
    s = lax.axis_index("s")

    pltpu.sync_copy(zz_hbm, acc_sh.at[pl.ds(s * RPT, RPT)])
    plsc.subcore_barrier()

    def _gather(i, buf, sem):
        pltpu.async_copy(hs_hbm.at[src_v.at[i]], buf, sem)

    def _gwait(i, buf, sem):
        pltpu.make_async_copy(hs_hbm.at[src_v.at[i]], buf, sem).wait()

    def _scat(i, buf, sem):
        pltpu.async_copy(buf, acc_sh.at[dst_v.at[i]], sem, add=True)

    def _swait(i, buf, sem):
        pltpu.make_async_copy(buf, acc_sh.at[dst_v.at[i]], sem).wait()

    @pl.loop(0, NSUPER)
    def _super(m):
        pltpu.sync_copy(src_hbm.at[s, m], src_v)
        pltpu.sync_copy(dst_hbm.at[s, m], dst_v)
        pltpu.sync_copy(ew_hbm.at[s, m], ew_v)

        @pl.loop(0, NCHUNK)
        def _adj(i):
            for j in range(CHUNK // 16):
                src_v[i, pl.ds(j * 16, 16)] = src_v[i, pl.ds(j * 16, 16)] + c * N

        _gather(0, bufA, gA)
        _gather(1, bufB, gB)

        @pl.loop(0, (NCHUNK - 1) // 3)
        def _it(t):
            e = t * 3
            _gwait(e, bufA, gA)
            _scale_chunk(bufA, ew_v, e)
            _scat(e, bufA, sA)

            @pl.when(t > 0)
            def _():
                _swait(e - 1, bufC, sC)
            _gather(e + 2, bufC, gC)

            _gwait(e + 1, bufB, gB)
            _scale_chunk(bufB, ew_v, e + 1)
            _scat(e + 1, bufB, sB)

            _swait(e, bufA, sA)
            _gather(e + 3, bufA, gA)

            _gwait(e + 2, bufC, gC)
            _scale_chunk(bufC, ew_v, e + 2)
            _scat(e + 2, bufC, sC)

            _swait(e + 1, bufB, sB)

            @pl.when(t < (NCHUNK - 1) // 3 - 1)
            def _():
                _gather(e + 4, bufB, gB)

        _swait(NCHUNK - 2, bufC, sC)
        _gwait(NCHUNK - 1, bufA, gA)
        _scale_chunk(bufA, ew_v, NCHUNK - 1)
        pltpu.sync_copy(bufA, acc_sh.at[dst_v.at[NCHUNK - 1]], add=True)

    plsc.subcore_barrier()
    pltpu.sync_copy(acc_sh.at[pl.ds(s * RPT, RPT)],
                    acc_hbm.at[c, pl.ds(s * RPT, RPT)])


@functools.partial(
    pl.kernel,
    out_type=jax.ShapeDtypeStruct((2, NPAD, H), jnp.float32),
    mesh=plsc.VectorSubcoreMesh(core_axis_name="c", subcore_axis_name="s"),
    scratch_types=[
        pltpu.VMEM_SHARED((NPAD, H), jnp.float32),
        pltpu.VMEM((NCHUNK, CHUNK), jnp.int32),
        pltpu.VMEM((NCHUNK, CHUNK), jnp.int32),
        pltpu.VMEM((SUPER,), jnp.float32),
        pltpu.VMEM((CHUNK, H), jnp.float32),
        pltpu.VMEM((CHUNK, H), jnp.float32),
        pltpu.VMEM((CHUNK, H), jnp.float32),
        pltpu.SemaphoreType.DMA,
        pltpu.SemaphoreType.DMA,
        pltpu.SemaphoreType.DMA,
        pltpu.SemaphoreType.DMA,
        pltpu.SemaphoreType.DMA,
        pltpu.SemaphoreType.DMA,
    ],
)
def _sc_aggregate(hs_hbm, src_hbm, dst_hbm, ew_hbm, zz_hbm, acc_hbm,
                  acc_sh, src_v, dst_v, ew_v, bufA, bufB, bufC,
                  gA, gB, gC, sA, sB, sC):
    _sc_body(hs_hbm, src_hbm, dst_hbm, ew_hbm, zz_hbm, acc_hbm,
             acc_sh, src_v, dst_v, ew_v, bufA, bufB, bufC,
             gA, gB, gC, sA, sB, sC)


DEG_EPT = E // 32
DEG_CH = 40


def _sc_deg_body(dst_hbm, ew_hbm, zz_hbm, deg_hbm, deg_sh, dst_v, ew_v):
    c = lax.axis_index("c")
    s = lax.axis_index("s")
    w = c * TILES + s

    pltpu.sync_copy(zz_hbm.at[pl.ds(s * RPT, RPT)],
                    deg_sh.at[pl.ds(s * RPT, RPT)])
    pltpu.sync_copy(dst_hbm.at[w], dst_v)
    pltpu.sync_copy(ew_hbm.at[w], ew_v)
    plsc.subcore_barrier()

    @pl.loop(0, DEG_CH)
    def _g(j):
        pltpu.sync_copy(ew_v.at[j], deg_sh.at[dst_v.at[j]], add=True)

    plsc.subcore_barrier()
    pltpu.sync_copy(deg_sh.at[pl.ds(s * RPT, RPT)],
                    deg_hbm.at[c, pl.ds(s * RPT, RPT)])


@functools.partial(
    pl.kernel,
    out_type=jax.ShapeDtypeStruct((2, NPAD), jnp.float32),
    mesh=plsc.VectorSubcoreMesh(core_axis_name="c", subcore_axis_name="s"),
    scratch_types=[
        pltpu.VMEM_SHARED((NPAD,), jnp.float32),
        pltpu.VMEM((DEG_CH, 128), jnp.int32),
        pltpu.VMEM((DEG_CH, 128), jnp.float32),
    ],
)
def _sc_degree(dst_hbm, ew_hbm, zz_hbm, deg_hbm, deg_sh, dst_v, ew_v):
    _sc_deg_body(dst_hbm, ew_hbm, zz_hbm, deg_hbm, deg_sh, dst_v, ew_v)


def kernel(x, edge_index, edge_attr, W1, b1, W2, b2, Wc1, bc1, Wc2, bc2, Wl, bl):
    src = edge_index[0].astype(jnp.int32)
    dst = edge_index[1].astype(jnp.int32)

    ew2d = _edge_weights(edge_attr, W1, b1, W2, b2)
    ewf = ew2d.reshape(E)

    dst_p = jnp.concatenate(
        [dst.reshape(32, DEG_EPT),
         jnp.full((32, DEG_CH * 128 - DEG_EPT), NPAD - 1, jnp.int32)], axis=1)
    ew_p = jnp.concatenate(
        [ewf.reshape(32, DEG_EPT),
         jnp.zeros((32, DEG_CH * 128 - DEG_EPT), jnp.float32)], axis=1)
    zzd = jnp.zeros((NPAD,), jnp.float32)
    degp = _sc_degree(dst_p.reshape(32, DEG_CH, 128),
                      ew_p.reshape(32, DEG_CH, 128), zzd)
    deg = degp[0, :N] + degp[1, :N] + 1.0
    dinv = (deg ** -0.5)[:, None]

    src3 = src.reshape(TILES, NSUPER, NCHUNK, CHUNK)
    dst3 = dst.reshape(TILES, NSUPER, NCHUNK, CHUNK)
    ew3 = ewf.reshape(TILES, NSUPER, SUPER)

    zz = jnp.zeros((RPT, H), jnp.float32)

    h1, hs1 = _pre1(x, Wc1.T, dinv)
    acc1 = _sc_aggregate(hs1.reshape(2 * N, H), src3, dst3, ew3, zz)
    h2, hs2 = _pre2(acc1, h1, dinv, bc1, Wc2.T)
    acc2 = _sc_aggregate(hs2.reshape(2 * N, H), src3, dst3, ew3, zz)
    return _final(acc2, h2, dinv, bc2, Wl.T, bl)


# speedup vs baseline: 1.2319x; 1.2319x over previous
import functools

import jax
import jax.numpy as jnp
from jax import lax
from jax.experimental import pallas as pl
from jax.experimental.pallas import tpu as pltpu
from jax.experimental.pallas import tpu_sc as plsc

N = 10000
NPAD = 10240
E = 160000
C = 256
H = 128
TILES = 16
EPT = E // TILES
CHUNK = 80
SUPER = 2000
NSUPER = EPT // SUPER
NCHUNK = SUPER // CHUNK
RPT = NPAD // TILES
ZROWS = 40


def _ew_kernel(eat_ref, w1_ref, b1_ref, w2_ref, b2_ref, o_ref):
    t = jnp.dot(w1_ref[...], eat_ref[...], preferred_element_type=jnp.float32)
    t = jnp.maximum(t + b1_ref[...], 0.0)
    v = jnp.sum(t * w2_ref[...], axis=0, keepdims=True)
    o_ref[...] = jnp.maximum(v + b2_ref[...], 0.0)


def _edge_weights(edge_attr, W1, b1, W2, b2):
    be = 3200
    return pl.pallas_call(
        _ew_kernel, grid=(E // be,),
        in_specs=[pl.BlockSpec((16, be), lambda i: (0, i)),
                  pl.BlockSpec((C, 16), lambda i: (0, 0)),
                  pl.BlockSpec((C, 1), lambda i: (0, 0)),
                  pl.BlockSpec((C, 1), lambda i: (0, 0)),
                  pl.BlockSpec((1, 1), lambda i: (0, 0))],
        out_specs=pl.BlockSpec((1, be), lambda i: (0, i)),
        out_shape=jax.ShapeDtypeStruct((1, E), jnp.float32),
    )(edge_attr.T, W1, b1[:, None], W2.T, b2[None, :])


def _pre1_kernel(x_ref, w_ref, dinv_ref, h_ref, hs_ref):
    h = jnp.dot(x_ref[...], w_ref[...], preferred_element_type=jnp.float32)
    h_ref[...] = h
    hs = h * dinv_ref[...]
    hs_ref[0] = hs[:, :H]
    hs_ref[1] = hs[:, H:]


def _pre1(x, Wt, dinv):
    bm = 2000
    return pl.pallas_call(
        _pre1_kernel, grid=(N // bm,),
        in_specs=[pl.BlockSpec((bm, C), lambda i: (i, 0)),
                  pl.BlockSpec((C, C), lambda i: (0, 0)),
                  pl.BlockSpec((bm, 1), lambda i: (i, 0))],
        out_specs=[pl.BlockSpec((bm, C), lambda i: (i, 0)),
                   pl.BlockSpec((2, bm, H), lambda i: (0, i, 0))],
        out_shape=[jax.ShapeDtypeStruct((N, C), jnp.float32),
                   jax.ShapeDtypeStruct((2, N, H), jnp.float32)],
    )(x, Wt, dinv)


def _pre2_kernel(a_ref, hp_ref, dinv_ref, b_ref, w_ref, h_ref, hs_ref):
    dinv = dinv_ref[...]
    a = jnp.concatenate([a_ref[0], a_ref[1]], axis=1)
    x2 = jnp.maximum(a * dinv + hp_ref[...] * (dinv * dinv) + b_ref[...], 0.0)
    h = jnp.dot(x2, w_ref[...], preferred_element_type=jnp.float32)
    h_ref[...] = h
    hs = h * dinv
    hs_ref[0] = hs[:, :H]
    hs_ref[1] = hs[:, H:]


def _pre2(acc, hprev, dinv, b, Wt):
    bm = 2000
    return pl.pallas_call(
        _pre2_kernel, grid=(N // bm,),
        in_specs=[pl.BlockSpec((2, bm, H), lambda i: (0, i, 0)),
                  pl.BlockSpec((bm, C), lambda i: (i, 0)),
                  pl.BlockSpec((bm, 1), lambda i: (i, 0)),
                  pl.BlockSpec((1, C), lambda i: (0, 0)),
                  pl.BlockSpec((C, C), lambda i: (0, 0))],
        out_specs=[pl.BlockSpec((bm, C), lambda i: (i, 0)),
                   pl.BlockSpec((2, bm, H), lambda i: (0, i, 0))],
        out_shape=[jax.ShapeDtypeStruct((N, C), jnp.float32),
                   jax.ShapeDtypeStruct((2, N, H), jnp.float32)],
    )(acc, hprev, dinv, b[None, :], Wt)


def _final_kernel(a_ref, hp_ref, dinv_ref, bc_ref, w_ref, bl_ref, o_ref):
    dinv = dinv_ref[...]
    a = jnp.concatenate([a_ref[0], a_ref[1]], axis=1)
    x3 = jnp.maximum(a * dinv + hp_ref[...] * (dinv * dinv) + bc_ref[...], 0.0)
    o_ref[...] = jnp.dot(x3, w_ref[...], preferred_element_type=jnp.float32) + bl_ref[...]


def _final(acc, hprev, dinv, bconv, Wt, bl):
    bm = 2000
    return pl.pallas_call(
        _final_kernel, grid=(N // bm,),
        in_specs=[pl.BlockSpec((2, bm, H), lambda i: (0, i, 0)),
                  pl.BlockSpec((bm, C), lambda i: (i, 0)),
                  pl.BlockSpec((bm, 1), lambda i: (i, 0)),
                  pl.BlockSpec((1, C), lambda i: (0, 0)),
                  pl.BlockSpec((C, C), lambda i: (0, 0)),
                  pl.BlockSpec((1, C), lambda i: (0, 0))],
        out_specs=pl.BlockSpec((bm, C), lambda i: (i, 0)),
        out_shape=jax.ShapeDtypeStruct((N, C), jnp.float32),
    )(acc, hprev, dinv, bconv[None, :], Wt, bl[None, :])


def _scale_chunk(rows, ew_v, i):
    @pl.loop(0, CHUNK // 16)
    def _scale(g):
        wvec = ew_v[pl.ds(i * CHUNK + g * 16, 16)]
        for k in range(16):
            w = lax.gather(
                wvec, jnp.full((16, 1), k, jnp.int32),
                lax.GatherDimensionNumbers(offset_dims=(),
                                           collapsed_slice_dims=(0,),
                                           start_index_map=(0,)),
                (1,), mode=lax.GatherScatterMode.PROMISE_IN_BOUNDS)
            r = g * 16 + k
            for j in range(H // 16):
                rows[r, pl.ds(j * 16, 16)] = rows[r, pl.ds(j * 16, 16)] * w


def _sc_body(hs_hbm, src_hbm, dst_hbm, ew_hbm, zz_hbm, acc_hbm,
             acc_sh, src_v, dst_v, ew_v, bufA, bufB, bufC,
             gA, gB, gC, sA, sB, sC):
    c = lax.axis_index("c")
    s = lax.axis_index("s")

    pltpu.sync_copy(zz_hbm, acc_sh.at[pl.ds(s * RPT, RPT)])
    plsc.subcore_barrier()

    def _gather(i, buf, sem):
        pltpu.async_copy(hs_hbm.at[src_v.at[i]], buf, sem)

    def _gwait(i, buf, sem):
        pltpu.make_async_copy(hs_hbm.at[src_v.at[i]], buf, sem).wait()

    def _scat(i, buf, sem):
        pltpu.async_copy(buf, acc_sh.at[dst_v.at[i]], sem, add=True)

    def _swait(i, buf, sem):
        pltpu.make_async_copy(buf, acc_sh.at[dst_v.at[i]], sem).wait()

    @pl.loop(0, NSUPER)
    def _super(m):
        pltpu.sync_copy(src_hbm.at[s, m], src_v)
        pltpu.sync_copy(dst_hbm.at[s, m], dst_v)
        pltpu.sync_copy(ew_hbm.at[s, m], ew_v)

        @pl.loop(0, NCHUNK)
        def _adj(i):
            for j in range(CHUNK // 16):
                src_v[i, pl.ds(j * 16, 16)] = src_v[i, pl.ds(j * 16, 16)] + c * N

        _gather(0, bufA, gA)
        _gather(1, bufB, gB)

        @pl.loop(0, (NCHUNK - 1) // 3)
        def _it(t):
            e = t * 3
            _gwait(e, bufA, gA)
            _scale_chunk(bufA, ew_v, e)
            _scat(e, bufA, sA)

            @pl.when(t > 0)
            def _():
                _swait(e - 1, bufC, sC)
            _gather(e + 2, bufC, gC)

            _gwait(e + 1, bufB, gB)
            _scale_chunk(bufB, ew_v, e + 1)
            _scat(e + 1, bufB, sB)

            _swait(e, bufA, sA)
            _gather(e + 3, bufA, gA)

            _gwait(e + 2, bufC, gC)
            _scale_chunk(bufC, ew_v, e + 2)
            _scat(e + 2, bufC, sC)

            _swait(e + 1, bufB, sB)

            @pl.when(t < (NCHUNK - 1) // 3 - 1)
            def _():
                _gather(e + 4, bufB, gB)

        _swait(NCHUNK - 2, bufC, sC)
        _gwait(NCHUNK - 1, bufA, gA)
        _scale_chunk(bufA, ew_v, NCHUNK - 1)
        pltpu.sync_copy(bufA, acc_sh.at[dst_v.at[NCHUNK - 1]], add=True)

    plsc.subcore_barrier()
    pltpu.sync_copy(acc_sh.at[pl.ds(s * RPT, RPT)],
                    acc_hbm.at[c, pl.ds(s * RPT, RPT)])


@functools.partial(
    pl.kernel,
    out_type=jax.ShapeDtypeStruct((2, NPAD, H), jnp.float32),
    mesh=plsc.VectorSubcoreMesh(core_axis_name="c", subcore_axis_name="s"),
    scratch_types=[
        pltpu.VMEM_SHARED((NPAD, H), jnp.float32),
        pltpu.VMEM((NCHUNK, CHUNK), jnp.int32),
        pltpu.VMEM((NCHUNK, CHUNK), jnp.int32),
        pltpu.VMEM((SUPER,), jnp.float32),
        pltpu.VMEM((CHUNK, H), jnp.float32),
        pltpu.VMEM((CHUNK, H), jnp.float32),
        pltpu.VMEM((CHUNK, H), jnp.float32),
        pltpu.SemaphoreType.DMA,
        pltpu.SemaphoreType.DMA,
        pltpu.SemaphoreType.DMA,
        pltpu.SemaphoreType.DMA,
        pltpu.SemaphoreType.DMA,
        pltpu.SemaphoreType.DMA,
    ],
)
def _sc_aggregate(hs_hbm, src_hbm, dst_hbm, ew_hbm, zz_hbm, acc_hbm,
                  acc_sh, src_v, dst_v, ew_v, bufA, bufB, bufC,
                  gA, gB, gC, sA, sB, sC):
    _sc_body(hs_hbm, src_hbm, dst_hbm, ew_hbm, zz_hbm, acc_hbm,
             acc_sh, src_v, dst_v, ew_v, bufA, bufB, bufC,
             gA, gB, gC, sA, sB, sC)


DEG_EPT = E // 32
DEG_CH = 40


def _sc_deg_body(dst_hbm, ew_hbm, zz_hbm, deg_hbm, deg_sh, dst_v, ew_v):
    c = lax.axis_index("c")
    s = lax.axis_index("s")
    w = c * TILES + s

    pltpu.sync_copy(zz_hbm.at[pl.ds(s * RPT, RPT)],
                    deg_sh.at[pl.ds(s * RPT, RPT)])
    pltpu.sync_copy(dst_hbm.at[w], dst_v)
    pltpu.sync_copy(ew_hbm.at[w], ew_v)
    plsc.subcore_barrier()

    @pl.loop(0, DEG_CH)
    def _g(j):
        pltpu.sync_copy(ew_v.at[j], deg_sh.at[dst_v.at[j]], add=True)

    plsc.subcore_barrier()
    pltpu.sync_copy(deg_sh.at[pl.ds(s * RPT, RPT)],
                    deg_hbm.at[c, pl.ds(s * RPT, RPT)])


@functools.partial(
    pl.kernel,
    out_type=jax.ShapeDtypeStruct((2, NPAD), jnp.float32),
    mesh=plsc.VectorSubcoreMesh(core_axis_name="c", subcore_axis_name="s"),
    scratch_types=[
        pltpu.VMEM_SHARED((NPAD,), jnp.float32),
        pltpu.VMEM((DEG_CH, 128), jnp.int32),
        pltpu.VMEM((DEG_CH, 128), jnp.float32),
    ],
)
def _sc_degree(dst_hbm, ew_hbm, zz_hbm, deg_hbm, deg_sh, dst_v, ew_v):
    _sc_deg_body(dst_hbm, ew_hbm, zz_hbm, deg_hbm, deg_sh, dst_v, ew_v)


def kernel(x, edge_index, edge_attr, W1, b1, W2, b2, Wc1, bc1, Wc2, bc2, Wl, bl):
    src = edge_index[0].astype(jnp.int32)
    dst = edge_index[1].astype(jnp.int32)

    ew2d = _edge_weights(edge_attr, W1, b1, W2, b2)
    ewf = ew2d.reshape(E)

    dst_p = jnp.concatenate(
        [dst.reshape(32, DEG_EPT),
         jnp.full((32, DEG_CH * 128 - DEG_EPT), NPAD - 1, jnp.int32)], axis=1)
    ew_p = jnp.concatenate(
        [ewf.reshape(32, DEG_EPT),
         jnp.zeros((32, DEG_CH * 128 - DEG_EPT), jnp.float32)], axis=1)
    zzd = jnp.zeros((NPAD,), jnp.float32)
    degp = _sc_degree(dst_p.reshape(32, DEG_CH, 128),
                      ew_p.reshape(32, DEG_CH, 128), zzd)
    deg = degp[0, :N] + degp[1, :N] + 1.0
    dinv = (deg ** -0.5)[:, None]

    src3 = src.reshape(TILES, NSUPER, NCHUNK, CHUNK)
    dst3 = dst.reshape(TILES, NSUPER, NCHUNK, CHUNK)
    ew3 = ewf.reshape(TILES, NSUPER, SUPER)

    zz = jnp.zeros((RPT, H), jnp.float32)

    h1, hs1 = _pre1(x, Wc1.T, dinv)
    acc1 = _sc_aggregate(hs1.reshape(2 * N, H), src3, dst3, ew3, zz)
    h2, hs2 = _pre2(acc1, h1, dinv, bc1, Wc2.T)
    acc2 = _sc_aggregate(hs2.reshape(2 * N, H), src3, dst3, ew3, zz)
    return _final(acc2, h2, dinv, bc2, Wl.T, bl)


# speedup vs baseline: 1.2498x; 1.0145x over previous
import functools

import jax
import jax.numpy as jnp
from jax import lax
from jax.experimental import pallas as pl
from jax.experimental.pallas import tpu as pltpu
from jax.experimental.pallas import tpu_sc as plsc

N = 10000
NPAD = 10240
E = 160000
C = 256
H = 128
TILES = 16
EPT = E // TILES
CHUNK = 80
SUPER = 2000
NSUPER = EPT // SUPER
NCHUNK = SUPER // CHUNK
RPT = NPAD // TILES
ZROWS = 40


def _ew_kernel(eat_ref, w1_ref, b1_ref, w2_ref, b2_ref, o_ref):
    t = jnp.dot(w1_ref[...], eat_ref[...], preferred_element_type=jnp.float32)
    t = jnp.maximum(t + b1_ref[...], 0.0)
    v = jnp.sum(t * w2_ref[...], axis=0, keepdims=True)
    o_ref[...] = jnp.maximum(v + b2_ref[...], 0.0)


def _edge_weights(edge_attr, W1, b1, W2, b2):
    be = 3200
    return pl.pallas_call(
        _ew_kernel, grid=(E // be,),
        in_specs=[pl.BlockSpec((16, be), lambda i: (0, i)),
                  pl.BlockSpec((C, 16), lambda i: (0, 0)),
                  pl.BlockSpec((C, 1), lambda i: (0, 0)),
                  pl.BlockSpec((C, 1), lambda i: (0, 0)),
                  pl.BlockSpec((1, 1), lambda i: (0, 0))],
        out_specs=pl.BlockSpec((1, be), lambda i: (0, i)),
        out_shape=jax.ShapeDtypeStruct((1, E), jnp.float32),
    )(edge_attr.T, W1, b1[:, None], W2.T, b2[None, :])


def _pre1_kernel(x_ref, w_ref, dinv_ref, h_ref, hs_ref):
    h = jnp.dot(x_ref[...], w_ref[...], preferred_element_type=jnp.float32)
    h_ref[...] = h
    hs = h * dinv_ref[...]
    hs_ref[0] = hs[:, :H]
    hs_ref[1] = hs[:, H:]


def _pre1(x, Wt, dinv):
    bm = 2000
    return pl.pallas_call(
        _pre1_kernel, grid=(N // bm,),
        in_specs=[pl.BlockSpec((bm, C), lambda i: (i, 0)),
                  pl.BlockSpec((C, C), lambda i: (0, 0)),
                  pl.BlockSpec((bm, 1), lambda i: (i, 0))],
        out_specs=[pl.BlockSpec((bm, C), lambda i: (i, 0)),
                   pl.BlockSpec((2, bm, H), lambda i: (0, i, 0))],
        out_shape=[jax.ShapeDtypeStruct((N, C), jnp.float32),
                   jax.ShapeDtypeStruct((2, N, H), jnp.float32)],
    )(x, Wt, dinv)


def _pre2_kernel(a_ref, hp_ref, dinv_ref, b_ref, w_ref, h_ref, hs_ref):
    dinv = dinv_ref[...]
    a = jnp.concatenate([a_ref[0], a_ref[1]], axis=1)
    x2 = jnp.maximum(a * dinv + hp_ref[...] * (dinv * dinv) + b_ref[...], 0.0)
    h = jnp.dot(x2, w_ref[...], preferred_element_type=jnp.float32)
    h_ref[...] = h
    hs = h * dinv
    hs_ref[0] = hs[:, :H]
    hs_ref[1] = hs[:, H:]


def _pre2(acc, hprev, dinv, b, Wt):
    bm = 2000
    return pl.pallas_call(
        _pre2_kernel, grid=(N // bm,),
        in_specs=[pl.BlockSpec((2, bm, H), lambda i: (0, i, 0)),
                  pl.BlockSpec((bm, C), lambda i: (i, 0)),
                  pl.BlockSpec((bm, 1), lambda i: (i, 0)),
                  pl.BlockSpec((1, C), lambda i: (0, 0)),
                  pl.BlockSpec((C, C), lambda i: (0, 0))],
        out_specs=[pl.BlockSpec((bm, C), lambda i: (i, 0)),
                   pl.BlockSpec((2, bm, H), lambda i: (0, i, 0))],
        out_shape=[jax.ShapeDtypeStruct((N, C), jnp.float32),
                   jax.ShapeDtypeStruct((2, N, H), jnp.float32)],
    )(acc, hprev, dinv, b[None, :], Wt)


def _final_kernel(a_ref, hp_ref, dinv_ref, bc_ref, w_ref, bl_ref, o_ref):
    dinv = dinv_ref[...]
    a = jnp.concatenate([a_ref[0], a_ref[1]], axis=1)
    x3 = jnp.maximum(a * dinv + hp_ref[...] * (dinv * dinv) + bc_ref[...], 0.0)
    o_ref[...] = jnp.dot(x3, w_ref[...], preferred_element_type=jnp.float32) + bl_ref[...]


def _final(acc, hprev, dinv, bconv, Wt, bl):
    bm = 2000
    return pl.pallas_call(
        _final_kernel, grid=(N // bm,),
        in_specs=[pl.BlockSpec((2, bm, H), lambda i: (0, i, 0)),
                  pl.BlockSpec((bm, C), lambda i: (i, 0)),
                  pl.BlockSpec((bm, 1), lambda i: (i, 0)),
                  pl.BlockSpec((1, C), lambda i: (0, 0)),
                  pl.BlockSpec((C, C), lambda i: (0, 0)),
                  pl.BlockSpec((1, C), lambda i: (0, 0))],
        out_specs=pl.BlockSpec((bm, C), lambda i: (i, 0)),
        out_shape=jax.ShapeDtypeStruct((N, C), jnp.float32),
    )(acc, hprev, dinv, bconv[None, :], Wt, bl[None, :])


def _scale_chunk(rows, ew_v, i):
    @pl.loop(0, CHUNK // 16)
    def _scale(g):
        wvec = ew_v[pl.ds(i * CHUNK + g * 16, 16)]
        for k in range(16):
            w = lax.gather(
                wvec, jnp.full((16, 1), k, jnp.int32),
                lax.GatherDimensionNumbers(offset_dims=(),
                                           collapsed_slice_dims=(0,),
                                           start_index_map=(0,)),
                (1,), mode=lax.GatherScatterMode.PROMISE_IN_BOUNDS)
            r = g * 16 + k
            for j in range(H // 16):
                rows[r, pl.ds(j * 16, 16)] = rows[r, pl.ds(j * 16, 16)] * w


def _sc_body(hs_hbm, src_hbm, dst_hbm, ew_hbm, zz_hbm, acc_hbm,
             acc_sh, src_v, dst_v, ew_v, bufA, bufB, bufC,
             gA, gB, gC, sA, sB, sC):
    c = lax.axis_index("c")
    s = lax.axis_index("s")

    pltpu.sync_copy(zz_hbm, acc_sh.at[pl.ds(s * RPT, RPT)])
    plsc.subcore_barrier()

    def _gather(i, buf, sem):
        pltpu.async_copy(hs_hbm.at[src_v.at[i]], buf, sem)

    def _gwait(i, buf, sem):
        pltpu.make_async_copy(hs_hbm.at[src_v.at[i]], buf, sem).wait()

    def _scat(i, buf, sem):
        pltpu.async_copy(buf, acc_sh.at[dst_v.at[i]], sem, add=True)

    def _swait(i, buf, sem):
        pltpu.make_async_copy(buf, acc_sh.at[dst_v.at[i]], sem).wait()

    @pl.loop(0, NSUPER)
    def _super(m):
        pltpu.sync_copy(src_hbm.at[c, s, m], src_v)
        pltpu.sync_copy(dst_hbm.at[s, m], dst_v)
        pltpu.sync_copy(ew_hbm.at[s, m], ew_v)

        _gather(0, bufA, gA)
        _gather(1, bufB, gB)

        @pl.loop(0, (NCHUNK - 1) // 3)
        def _it(t):
            e = t * 3
            _gwait(e, bufA, gA)
            _scale_chunk(bufA, ew_v, e)
            _scat(e, bufA, sA)

            @pl.when(t > 0)
            def _():
                _swait(e - 1, bufC, sC)
            _gather(e + 2, bufC, gC)

            _gwait(e + 1, bufB, gB)
            _scale_chunk(bufB, ew_v, e + 1)
            _scat(e + 1, bufB, sB)

            _swait(e, bufA, sA)
            _gather(e + 3, bufA, gA)

            _gwait(e + 2, bufC, gC)
            _scale_chunk(bufC, ew_v, e + 2)
            _scat(e + 2, bufC, sC)

            _swait(e + 1, bufB, sB)

            @pl.when(t < (NCHUNK - 1) // 3 - 1)
            def _():
                _gather(e + 4, bufB, gB)

        _swait(NCHUNK - 2, bufC, sC)
        _gwait(NCHUNK - 1, bufA, gA)
        _scale_chunk(bufA, ew_v, NCHUNK - 1)
        pltpu.sync_copy(bufA, acc_sh.at[dst_v.at[NCHUNK - 1]], add=True)

    plsc.subcore_barrier()
    pltpu.sync_copy(acc_sh.at[pl.ds(s * RPT, RPT)],
                    acc_hbm.at[c, pl.ds(s * RPT, RPT)])


@functools.partial(
    pl.kernel,
    out_type=jax.ShapeDtypeStruct((2, NPAD, H), jnp.float32),
    mesh=plsc.VectorSubcoreMesh(core_axis_name="c", subcore_axis_name="s"),
    scratch_types=[
        pltpu.VMEM_SHARED((NPAD, H), jnp.float32),
        pltpu.VMEM((NCHUNK, CHUNK), jnp.int32),
        pltpu.VMEM((NCHUNK, CHUNK), jnp.int32),
        pltpu.VMEM((SUPER,), jnp.float32),
        pltpu.VMEM((CHUNK, H), jnp.float32),
        pltpu.VMEM((CHUNK, H), jnp.float32),
        pltpu.VMEM((CHUNK, H), jnp.float32),
        pltpu.SemaphoreType.DMA,
        pltpu.SemaphoreType.DMA,
        pltpu.SemaphoreType.DMA,
        pltpu.SemaphoreType.DMA,
        pltpu.SemaphoreType.DMA,
        pltpu.SemaphoreType.DMA,
    ],
)
def _sc_aggregate(hs_hbm, src_hbm, dst_hbm, ew_hbm, zz_hbm, acc_hbm,
                  acc_sh, src_v, dst_v, ew_v, bufA, bufB, bufC,
                  gA, gB, gC, sA, sB, sC):
    _sc_body(hs_hbm, src_hbm, dst_hbm, ew_hbm, zz_hbm, acc_hbm,
             acc_sh, src_v, dst_v, ew_v, bufA, bufB, bufC,
             gA, gB, gC, sA, sB, sC)


DEG_EPT = E // 32
DEG_CH = 40


def _sc_deg_body(dst_hbm, ew_hbm, zz_hbm, deg_hbm, deg_sh, dst_v, ew_v):
    c = lax.axis_index("c")
    s = lax.axis_index("s")
    w = c * TILES + s

    pltpu.sync_copy(zz_hbm.at[pl.ds(s * RPT, RPT)],
                    deg_sh.at[pl.ds(s * RPT, RPT)])
    pltpu.sync_copy(dst_hbm.at[w], dst_v)
    pltpu.sync_copy(ew_hbm.at[w], ew_v)
    plsc.subcore_barrier()

    @pl.loop(0, DEG_CH)
    def _g(j):
        pltpu.sync_copy(ew_v.at[j], deg_sh.at[dst_v.at[j]], add=True)

    plsc.subcore_barrier()
    pltpu.sync_copy(deg_sh.at[pl.ds(s * RPT, RPT)],
                    deg_hbm.at[c, pl.ds(s * RPT, RPT)])


@functools.partial(
    pl.kernel,
    out_type=jax.ShapeDtypeStruct((2, NPAD), jnp.float32),
    mesh=plsc.VectorSubcoreMesh(core_axis_name="c", subcore_axis_name="s"),
    scratch_types=[
        pltpu.VMEM_SHARED((NPAD,), jnp.float32),
        pltpu.VMEM((DEG_CH, 128), jnp.int32),
        pltpu.VMEM((DEG_CH, 128), jnp.float32),
    ],
)
def _sc_degree(dst_hbm, ew_hbm, zz_hbm, deg_hbm, deg_sh, dst_v, ew_v):
    _sc_deg_body(dst_hbm, ew_hbm, zz_hbm, deg_hbm, deg_sh, dst_v, ew_v)


def kernel(x, edge_index, edge_attr, W1, b1, W2, b2, Wc1, bc1, Wc2, bc2, Wl, bl):
    src = edge_index[0].astype(jnp.int32)
    dst = edge_index[1].astype(jnp.int32)

    ew2d = _edge_weights(edge_attr, W1, b1, W2, b2)
    ewf = ew2d.reshape(E)

    dst_p = jnp.concatenate(
        [dst.reshape(32, DEG_EPT),
         jnp.full((32, DEG_CH * 128 - DEG_EPT), NPAD - 1, jnp.int32)], axis=1)
    ew_p = jnp.concatenate(
        [ewf.reshape(32, DEG_EPT),
         jnp.zeros((32, DEG_CH * 128 - DEG_EPT), jnp.float32)], axis=1)
    zzd = jnp.zeros((NPAD,), jnp.float32)
    degp = _sc_degree(dst_p.reshape(32, DEG_CH, 128),
                      ew_p.reshape(32, DEG_CH, 128), zzd)
    deg = degp[0, :N] + degp[1, :N] + 1.0
    dinv = (deg ** -0.5)[:, None]

    src3 = src.reshape(TILES, NSUPER, NCHUNK, CHUNK)
    src3 = jnp.stack([src3, src3 + N])
    dst3 = dst.reshape(TILES, NSUPER, NCHUNK, CHUNK)
    ew3 = ewf.reshape(TILES, NSUPER, SUPER)

    zz = jnp.zeros((RPT, H), jnp.float32)

    h1, hs1 = _pre1(x, Wc1.T, dinv)
    acc1 = _sc_aggregate(hs1.reshape(2 * N, H), src3, dst3, ew3, zz)
    h2, hs2 = _pre2(acc1, h1, dinv, bc1, Wc2.T)
    acc2 = _sc_aggregate(hs2.reshape(2 * N, H), src3, dst3, ew3, zz)
    return _final(acc2, h2, dinv, bc2, Wl.T, bl)
